# Initial kernel scaffold; baseline (speedup 1.0000x reference)
import jax
import jax.numpy as jnp
from jax.experimental import pallas as pl


def kernel(x, gate_w, Wg, Wu, Wd, SWg, SWu, SWd):
    raise NotImplementedError("write your pallas kernel here")


# speedup vs baseline: 1.2547x; 1.2547x over previous
import functools

import jax
import jax.numpy as jnp
from jax.experimental import pallas as pl
from jax.experimental.pallas import tpu as pltpu

E = 16
TOP_K = 2
H = 768
F = 2048
T = 64


def _moe_ffn_kernel(x_ref, gw_ref, wg_ref, wu_ref, wd_ref, out_ref, comb_ref):
    e = pl.program_id(0)

    @pl.when(e == 0)
    def _gating():
        x = x_ref[...]
        logits = jax.lax.dot_general(
            x, gw_ref[...], (((1,), (1,)), ((), ())),
            preferred_element_type=jnp.float32)
        m = jnp.max(logits, axis=-1, keepdims=True)
        ex = jnp.exp(logits - m)
        scores = ex / jnp.sum(ex, axis=-1, keepdims=True)
        iota = jax.lax.broadcasted_iota(jnp.int32, (T, E), 1)
        m1 = jnp.max(scores, axis=-1, keepdims=True)
        i1 = jnp.min(jnp.where(scores == m1, iota, E), axis=-1, keepdims=True)
        masked = jnp.where(iota == i1, -jnp.inf, scores)
        m2 = jnp.max(masked, axis=-1, keepdims=True)
        i2 = jnp.min(jnp.where(masked == m2, iota, E), axis=-1, keepdims=True)
        denom = m1 + m2 + 1e-20
        comb = jnp.where(iota == i1, m1 / denom, 0.0)
        comb = comb + jnp.where(iota == i2, m2 / denom, 0.0)
        comb_ref[...] = comb
        out_ref[...] = jnp.zeros_like(out_ref)

    x = x_ref[...]
    g = jnp.dot(x, wg_ref[0], preferred_element_type=jnp.float32)
    u = jnp.dot(x, wu_ref[0], preferred_element_type=jnp.float32)
    act = g * jax.lax.logistic(g) * u
    o = jnp.dot(act, wd_ref[0], preferred_element_type=jnp.float32)
    lane = jax.lax.broadcasted_iota(jnp.int32, (T, E), 1)
    w_col = jnp.sum(jnp.where(lane == e, comb_ref[...], 0.0),
                    axis=-1, keepdims=True)
    out_ref[...] += w_col * o


def _shared_kernel(x_ref, swg_ref, swu_ref, swd_ref, y_ref, out_ref):
    x = x_ref[...]
    g = jnp.dot(x, swg_ref[...], preferred_element_type=jnp.float32)
    u = jnp.dot(x, swu_ref[...], preferred_element_type=jnp.float32)
    act = g * jax.lax.logistic(g) * u
    out_ref[...] = y_ref[...] + jnp.dot(act, swd_ref[...],
                                        preferred_element_type=jnp.float32)


@jax.jit
def kernel(x, gate_w, Wg, Wu, Wd, SWg, SWu, SWd):
    b, s, h = x.shape
    x2 = x.reshape(-1, h)

    y = pl.pallas_call(
        _moe_ffn_kernel,
        grid=(E,),
        in_specs=[
            pl.BlockSpec((T, H), lambda e: (0, 0)),
            pl.BlockSpec((E, H), lambda e: (0, 0)),
            pl.BlockSpec((1, H, F), lambda e: (e, 0, 0)),
            pl.BlockSpec((1, H, F), lambda e: (e, 0, 0)),
            pl.BlockSpec((1, F, H), lambda e: (e, 0, 0)),
        ],
        out_specs=pl.BlockSpec((T, H), lambda e: (0, 0)),
        out_shape=jax.ShapeDtypeStruct((T, H), jnp.float32),
        scratch_shapes=[pltpu.VMEM((T, E), jnp.float32)],
    )(x2, gate_w, Wg, Wu, Wd)

    out = pl.pallas_call(
        _shared_kernel,
        out_shape=jax.ShapeDtypeStruct((T, H), jnp.float32),
    )(x2, SWg, SWu, SWd, y)

    return out.reshape(b, s, h)


# speedup vs baseline: 1.3049x; 1.0400x over previous
import jax
import jax.numpy as jnp
from jax.experimental import pallas as pl
from jax.experimental.pallas import tpu as pltpu

E = 16
H = 768
F = 2048
T = 64
NCH = 4
FC = F // NCH
NBUF = 6
NTILES = (E + 1) * NCH


def _gating(x, gw):
    logits = jax.lax.dot_general(
        x, gw, (((1,), (1,)), ((), ())),
        preferred_element_type=jnp.float32)
    m = jnp.max(logits, axis=-1, keepdims=True)
    ex = jnp.exp(logits - m)
    scores = ex / jnp.sum(ex, axis=-1, keepdims=True)
    iota = jax.lax.broadcasted_iota(jnp.int32, (T, E), 1)
    m1 = jnp.max(scores, axis=-1, keepdims=True)
    i1 = jnp.min(jnp.where(scores == m1, iota, E), axis=-1, keepdims=True)
    masked = jnp.where(iota == i1, -jnp.inf, scores)
    m2 = jnp.max(masked, axis=-1, keepdims=True)
    i2 = jnp.min(jnp.where(masked == m2, iota, E), axis=-1, keepdims=True)
    denom = m1 + m2 + 1e-20
    comb = jnp.where(iota == i1, m1 / denom, 0.0)
    return comb + jnp.where(iota == i2, m2 / denom, 0.0)


def _ffn_kernel(x_ref, gw_ref, wg_hbm, wu_hbm, wd_hbm, swg_hbm, swu_hbm,
                swd_hbm, out_ref, wg_buf, wu_buf, wd_buf, comb_ref, sem):

    def issue(t, slot):
        e = t // NCH
        f0 = (t % NCH) * FC

        @pl.when(e < E)
        def _():
            pltpu.make_async_copy(
                wg_hbm.at[e, :, pl.ds(f0, FC)], wg_buf.at[slot],
                sem.at[0, slot]).start()
            pltpu.make_async_copy(
                wu_hbm.at[e, :, pl.ds(f0, FC)], wu_buf.at[slot],
                sem.at[1, slot]).start()
            pltpu.make_async_copy(
                wd_hbm.at[e, pl.ds(f0, FC), :], wd_buf.at[slot],
                sem.at[2, slot]).start()

        @pl.when(e == E)
        def _():
            pltpu.make_async_copy(
                swg_hbm.at[:, pl.ds(f0, FC)], wg_buf.at[slot],
                sem.at[0, slot]).start()
            pltpu.make_async_copy(
                swu_hbm.at[:, pl.ds(f0, FC)], wu_buf.at[slot],
                sem.at[1, slot]).start()
            pltpu.make_async_copy(
                swd_hbm.at[pl.ds(f0, FC), :], wd_buf.at[slot],
                sem.at[2, slot]).start()

    def wait(slot):
        pltpu.make_async_copy(
            wg_hbm.at[0, :, pl.ds(0, FC)], wg_buf.at[slot],
            sem.at[0, slot]).wait()
        pltpu.make_async_copy(
            wu_hbm.at[0, :, pl.ds(0, FC)], wu_buf.at[slot],
            sem.at[1, slot]).wait()
        pltpu.make_async_copy(
            wd_hbm.at[0, pl.ds(0, FC), :], wd_buf.at[slot],
            sem.at[2, slot]).wait()

    for t in range(NBUF - 1):
        issue(jnp.int32(t), jnp.int32(t))

    comb_ref[...] = _gating(x_ref[...], gw_ref[...])
    out_ref[...] = jnp.zeros_like(out_ref)

    def body(t, _):
        slot = jax.lax.rem(t, NBUF)
        wait(slot)
        nxt = t + NBUF - 1

        @pl.when(nxt < NTILES)
        def _():
            issue(nxt, jax.lax.rem(nxt, NBUF))

        e = t // NCH
        x = x_ref[...]
        g = jnp.dot(x, wg_buf[slot], preferred_element_type=jnp.float32)
        u = jnp.dot(x, wu_buf[slot], preferred_element_type=jnp.float32)
        act = g * jax.lax.logistic(g) * u
        o = jnp.dot(act, wd_buf[slot], preferred_element_type=jnp.float32)
        lane = jax.lax.broadcasted_iota(jnp.int32, (T, E), 1)
        w_col = jnp.sum(jnp.where(lane == e, comb_ref[...], 0.0),
                        axis=-1, keepdims=True)
        w_col = w_col + jnp.where(e == E, 1.0, 0.0)
        out_ref[...] += w_col * o
        return 0

    jax.lax.fori_loop(0, NTILES, body, 0)


@jax.jit
def kernel(x, gate_w, Wg, Wu, Wd, SWg, SWu, SWd):
    b, s, h = x.shape
    x2 = x.reshape(-1, h)

    out = pl.pallas_call(
        _ffn_kernel,
        in_specs=[
            pl.BlockSpec(memory_space=pltpu.MemorySpace.VMEM),
            pl.BlockSpec(memory_space=pltpu.MemorySpace.VMEM),
            pl.BlockSpec(memory_space=pltpu.MemorySpace.HBM),
            pl.BlockSpec(memory_space=pltpu.MemorySpace.HBM),
            pl.BlockSpec(memory_space=pltpu.MemorySpace.HBM),
            pl.BlockSpec(memory_space=pltpu.MemorySpace.HBM),
            pl.BlockSpec(memory_space=pltpu.MemorySpace.HBM),
            pl.BlockSpec(memory_space=pltpu.MemorySpace.HBM),
        ],
        out_specs=pl.BlockSpec(memory_space=pltpu.MemorySpace.VMEM),
        out_shape=jax.ShapeDtypeStruct((T, H), jnp.float32),
        scratch_shapes=[
            pltpu.VMEM((NBUF, H, FC), jnp.float32),
            pltpu.VMEM((NBUF, H, FC), jnp.float32),
            pltpu.VMEM((NBUF, FC, H), jnp.float32),
            pltpu.VMEM((T, E), jnp.float32),
            pltpu.SemaphoreType.DMA((3, NBUF)),
        ],
    )(x2, gate_w, Wg, Wu, Wd, SWg, SWu, SWd)

    return out.reshape(b, s, h)


# speedup vs baseline: 1.3122x; 1.0056x over previous
import jax
import jax.numpy as jnp
from jax.experimental import pallas as pl
from jax.experimental.pallas import tpu as pltpu

E = 16
H = 768
F = 2048
T = 64
NCH = 2
FC = F // NCH
NBUF = 4
NTILES = (E + 1) * NCH


def _gating(x, gw):
    logits = jax.lax.dot_general(
        x, gw, (((1,), (1,)), ((), ())),
        preferred_element_type=jnp.float32)
    m = jnp.max(logits, axis=-1, keepdims=True)
    ex = jnp.exp(logits - m)
    scores = ex / jnp.sum(ex, axis=-1, keepdims=True)
    iota = jax.lax.broadcasted_iota(jnp.int32, (T, E), 1)
    m1 = jnp.max(scores, axis=-1, keepdims=True)
    i1 = jnp.min(jnp.where(scores == m1, iota, E), axis=-1, keepdims=True)
    masked = jnp.where(iota == i1, -jnp.inf, scores)
    m2 = jnp.max(masked, axis=-1, keepdims=True)
    i2 = jnp.min(jnp.where(masked == m2, iota, E), axis=-1, keepdims=True)
    denom = m1 + m2 + 1e-20
    comb = jnp.where(iota == i1, m1 / denom, 0.0)
    return comb + jnp.where(iota == i2, m2 / denom, 0.0)


def _ffn_kernel(x_ref, gw_ref, wg_hbm, wu_hbm, wd_hbm, swg_hbm, swu_hbm,
                swd_hbm, out_ref, wg_buf, wu_buf, wd_buf, comb_ref, sem):

    def issue(t, slot):
        e = t // NCH
        f0 = (t % NCH) * FC

        @pl.when(e < E)
        def _():
            pltpu.make_async_copy(
                wg_hbm.at[e, :, pl.ds(f0, FC)], wg_buf.at[slot],
                sem.at[0, slot]).start()
            pltpu.make_async_copy(
                wu_hbm.at[e, :, pl.ds(f0, FC)], wu_buf.at[slot],
                sem.at[1, slot]).start()
            pltpu.make_async_copy(
                wd_hbm.at[e, pl.ds(f0, FC), :], wd_buf.at[slot],
                sem.at[2, slot]).start()

        @pl.when(e == E)
        def _():
            pltpu.make_async_copy(
                swg_hbm.at[:, pl.ds(f0, FC)], wg_buf.at[slot],
                sem.at[0, slot]).start()
            pltpu.make_async_copy(
                swu_hbm.at[:, pl.ds(f0, FC)], wu_buf.at[slot],
                sem.at[1, slot]).start()
            pltpu.make_async_copy(
                swd_hbm.at[pl.ds(f0, FC), :], wd_buf.at[slot],
                sem.at[2, slot]).start()

    def wait(slot):
        pltpu.make_async_copy(
            wg_hbm.at[0, :, pl.ds(0, FC)], wg_buf.at[slot],
            sem.at[0, slot]).wait()
        pltpu.make_async_copy(
            wu_hbm.at[0, :, pl.ds(0, FC)], wu_buf.at[slot],
            sem.at[1, slot]).wait()
        pltpu.make_async_copy(
            wd_hbm.at[0, pl.ds(0, FC), :], wd_buf.at[slot],
            sem.at[2, slot]).wait()

    for t in range(NBUF - 1):
        issue(jnp.int32(t), jnp.int32(t))

    comb_ref[...] = _gating(x_ref[...], gw_ref[...])
    out_ref[...] = jnp.zeros_like(out_ref)

    def body(t, _):
        slot = jax.lax.rem(t, NBUF)
        wait(slot)
        nxt = t + NBUF - 1

        @pl.when(nxt < NTILES)
        def _():
            issue(nxt, jax.lax.rem(nxt, NBUF))

        e = t // NCH
        x = x_ref[...]
        g = jnp.dot(x, wg_buf[slot], preferred_element_type=jnp.float32)
        u = jnp.dot(x, wu_buf[slot], preferred_element_type=jnp.float32)
        act = g * jax.lax.logistic(g) * u
        o = jnp.dot(act, wd_buf[slot], preferred_element_type=jnp.float32)
        lane = jax.lax.broadcasted_iota(jnp.int32, (T, E), 1)
        w_col = jnp.sum(jnp.where(lane == e, comb_ref[...], 0.0),
                        axis=-1, keepdims=True)
        w_col = w_col + jnp.where(e == E, 1.0, 0.0)
        out_ref[...] += w_col * o
        return 0

    jax.lax.fori_loop(0, NTILES, body, 0)


@jax.jit
def kernel(x, gate_w, Wg, Wu, Wd, SWg, SWu, SWd):
    b, s, h = x.shape
    x2 = x.reshape(-1, h)

    out = pl.pallas_call(
        _ffn_kernel,
        in_specs=[
            pl.BlockSpec(memory_space=pltpu.MemorySpace.VMEM),
            pl.BlockSpec(memory_space=pltpu.MemorySpace.VMEM),
            pl.BlockSpec(memory_space=pltpu.MemorySpace.HBM),
            pl.BlockSpec(memory_space=pltpu.MemorySpace.HBM),
            pl.BlockSpec(memory_space=pltpu.MemorySpace.HBM),
            pl.BlockSpec(memory_space=pltpu.MemorySpace.HBM),
            pl.BlockSpec(memory_space=pltpu.MemorySpace.HBM),
            pl.BlockSpec(memory_space=pltpu.MemorySpace.HBM),
        ],
        out_specs=pl.BlockSpec(memory_space=pltpu.MemorySpace.VMEM),
        out_shape=jax.ShapeDtypeStruct((T, H), jnp.float32),
        scratch_shapes=[
            pltpu.VMEM((NBUF, H, FC), jnp.float32),
            pltpu.VMEM((NBUF, H, FC), jnp.float32),
            pltpu.VMEM((NBUF, FC, H), jnp.float32),
            pltpu.VMEM((T, E), jnp.float32),
            pltpu.SemaphoreType.DMA((3, NBUF)),
        ],
    )(x2, gate_w, Wg, Wu, Wd, SWg, SWu, SWd)

    return out.reshape(b, s, h)
